# baseline (speedup 1.0000x reference)
import math

import jax
import jax.numpy as jnp
from jax import lax
from jax.experimental import pallas as pl
from jax.experimental.pallas import tpu as pltpu

E = 128
N = 64
S = 8192
SB = 256
TAU = 1.0
THRESHOLD = 0.2

_HI = lax.Precision.HIGHEST


def _col_bcast(row, ones_row):
    return lax.dot_general(row, ones_row, (((0,), (0,)), ((), ())),
                           preferred_element_type=jnp.float32, precision=_HI)


def _attn_kernel(t_ref, wq_ref, bq_ref, wk_ref, wv_ref, bv_ref, wo_ref,
                 bo_ref, g_ref, c_ref, out_ref, mask_ref,
                 a_ref, m_ref, d_ref, cv_ref, l_ref):
    i = pl.program_id(0)
    nb = pl.num_programs(0)
    scale = 1.0 / math.sqrt(E)

    @pl.when(i == 0)
    def _init():
        q = lax.dot_general(t_ref[...], wq_ref[...], (((1,), (1,)), ((), ())),
                            preferred_element_type=jnp.float32,
                            precision=_HI) + bq_ref[...]
        a_ref[...] = lax.dot_general(q * scale, wk_ref[...],
                                     (((1,), (0,)), ((), ())),
                                     preferred_element_type=jnp.float32,
                                     precision=_HI)
        m_ref[...] = jnp.full((1, N), -jnp.inf, jnp.float32)
        d_ref[...] = jnp.zeros((1, N), jnp.float32)
        cv_ref[...] = jnp.zeros((N, E), jnp.float32)

    c = c_ref[...]
    c2 = c.reshape(SB * N, E)
    z = lax.dot_general(c2, a_ref[...], (((1,), (1,)), ((), ())),
                        preferred_element_type=jnp.float32,
                        precision=None)
    z3 = z.reshape(SB, N, N)
    row = lax.broadcasted_iota(jnp.int32, (1, N, N), 1)
    col = lax.broadcasted_iota(jnp.int32, (1, N, N), 2)
    eye = (row == col).astype(jnp.float32)
    logits = jnp.sum(z3 * eye, axis=-1)
    l_ref[pl.ds(i * SB, SB), :] = logits

    ones_row = jnp.ones((1, E), jnp.float32)
    m_old = m_ref[...]
    m_new = jnp.maximum(m_old, jnp.max(logits, axis=0, keepdims=True))
    corr = jnp.exp(m_old - m_new)
    p = jnp.exp(logits - m_new)
    m_ref[...] = m_new
    d_ref[...] = d_ref[...] * corr + jnp.sum(p, axis=0, keepdims=True)

    c3 = c.reshape(SB, N * E)
    r = lax.dot_general(p, c3, (((0,), (0,)), ((), ())),
                        preferred_element_type=jnp.float32,
                        precision=None)
    contrib = jnp.concatenate(
        [r[n:n + 1, n * E:(n + 1) * E] for n in range(N)], axis=0)
    cv_ref[...] = cv_ref[...] * _col_bcast(corr, ones_row) + contrib

    @pl.when(i == nb - 1)
    def _finish():
        m = m_ref[...]
        inv_d = 1.0 / d_ref[...]
        cv = cv_ref[...] * _col_bcast(inv_d, ones_row)
        ctx = lax.dot_general(cv, wv_ref[...], (((1,), (1,)), ((), ())),
                              preferred_element_type=jnp.float32,
                              precision=_HI) + bv_ref[...]
        out_ref[...] = lax.dot_general(ctx, wo_ref[...],
                                       (((1,), (1,)), ((), ())),
                                       preferred_element_type=jnp.float32,
                                       precision=_HI) + bo_ref[...]
        attn = jnp.exp(l_ref[...] - m) * inv_d
        zz = (attn + g_ref[...]) / TAU
        y = jnp.exp(zz - jnp.max(zz, axis=0, keepdims=True))
        y_soft = y / jnp.sum(y, axis=0, keepdims=True)
        mask_ref[...] = (y_soft > THRESHOLD).astype(jnp.int8)


@jax.jit
def kernel(target_emb, candidate_emb, in_proj_weight, in_proj_bias,
           out_proj_weight, out_proj_bias):
    t = target_emb[0]
    wq = in_proj_weight[:E]
    wk = in_proj_weight[E:2 * E]
    wv = in_proj_weight[2 * E:]
    wo = out_proj_weight
    bq = in_proj_bias[:E].reshape(1, E)
    bv = in_proj_bias[2 * E:].reshape(1, E)
    bo = out_proj_bias.reshape(1, E)

    u = jax.random.uniform(jax.random.key(42), (N, 1, S),
                           minval=1e-10, maxval=1.0)
    g = -jnp.log(-jnp.log(u))
    g_t = g[:, 0, :].T

    nb = S // SB
    out, mask = pl.pallas_call(
        _attn_kernel,
        grid=(nb,),
        in_specs=[
            pl.BlockSpec((N, E), lambda i: (0, 0)),
            pl.BlockSpec((E, E), lambda i: (0, 0)),
            pl.BlockSpec((1, E), lambda i: (0, 0)),
            pl.BlockSpec((E, E), lambda i: (0, 0)),
            pl.BlockSpec((E, E), lambda i: (0, 0)),
            pl.BlockSpec((1, E), lambda i: (0, 0)),
            pl.BlockSpec((E, E), lambda i: (0, 0)),
            pl.BlockSpec((1, E), lambda i: (0, 0)),
            pl.BlockSpec((S, N), lambda i: (0, 0)),
            pl.BlockSpec((SB, N, E), lambda i: (i, 0, 0)),
        ],
        out_specs=[
            pl.BlockSpec((N, E), lambda i: (0, 0)),
            pl.BlockSpec((S, N), lambda i: (0, 0)),
        ],
        out_shape=[
            jax.ShapeDtypeStruct((N, E), jnp.float32),
            jax.ShapeDtypeStruct((S, N), jnp.int8),
        ],
        scratch_shapes=[
            pltpu.VMEM((N, E), jnp.float32),
            pltpu.VMEM((1, N), jnp.float32),
            pltpu.VMEM((1, N), jnp.float32),
            pltpu.VMEM((N, E), jnp.float32),
            pltpu.VMEM((S, N), jnp.float32),
        ],
    )(t, wq, bq, wk, wv, bv, wo, bo, g_t, candidate_emb)

    attn_output = out
    candidate_mask = mask.T.astype(jnp.bool_).reshape(N, 1, S)
    return attn_output, candidate_mask


# speedup vs baseline: 1.3476x; 1.3476x over previous
import math

import jax
import jax.numpy as jnp
from jax import lax
from jax.experimental import pallas as pl
from jax.experimental.pallas import tpu as pltpu

E = 128
N = 64
S = 8192
SB = 256
TAU = 1.0
THRESHOLD = 0.2

_HI = lax.Precision.HIGHEST


def _col_bcast(row, ones_row):
    return lax.dot_general(row, ones_row, (((0,), (0,)), ((), ())),
                           preferred_element_type=jnp.float32, precision=_HI)


def _attn_kernel(t_ref, wq_ref, bq_ref, wk_ref, wv_ref, bv_ref, wo_ref,
                 bo_ref, g_ref, c_ref, out_ref, mask_ref,
                 a_ref, d_ref, cv_ref, l_ref):
    i = pl.program_id(0)
    nb = pl.num_programs(0)
    scale = 1.0 / math.sqrt(E)

    @pl.when(i == 0)
    def _init():
        q = lax.dot_general(t_ref[...], wq_ref[...], (((1,), (1,)), ((), ())),
                            preferred_element_type=jnp.float32,
                            precision=_HI) + bq_ref[...]
        a_ref[...] = lax.dot_general(q * scale, wk_ref[...],
                                     (((1,), (0,)), ((), ())),
                                     preferred_element_type=jnp.float32,
                                     precision=_HI)
        d_ref[...] = jnp.zeros((1, N), jnp.float32)
        cv_ref[...] = jnp.zeros((N, E), jnp.float32)

    c = c_ref[...]
    logits = jnp.sum(c * a_ref[...][None, :, :], axis=-1)
    l_ref[pl.ds(i * SB, SB), :] = logits

    p = jnp.exp(logits)
    d_ref[...] += jnp.sum(p, axis=0, keepdims=True)

    c3 = c.reshape(SB, N * E).astype(jnp.bfloat16)
    p16 = p.astype(jnp.bfloat16)
    r = lax.dot_general(p16, c3, (((0,), (0,)), ((), ())),
                        preferred_element_type=jnp.float32)
    contrib = jnp.concatenate(
        [r[n:n + 1, n * E:(n + 1) * E] for n in range(N)], axis=0)
    cv_ref[...] += contrib

    @pl.when(i == nb - 1)
    def _finish():
        inv_d = 1.0 / d_ref[...]
        ones_row = jnp.ones((1, E), jnp.float32)
        cv = cv_ref[...] * _col_bcast(inv_d, ones_row)
        ctx = lax.dot_general(cv, wv_ref[...], (((1,), (1,)), ((), ())),
                              preferred_element_type=jnp.float32,
                              precision=_HI) + bv_ref[...]
        out_ref[...] = lax.dot_general(ctx, wo_ref[...],
                                       (((1,), (1,)), ((), ())),
                                       preferred_element_type=jnp.float32,
                                       precision=_HI) + bo_ref[...]
        attn = jnp.exp(l_ref[...]) * inv_d
        zz = (attn + g_ref[...]) / TAU
        y = jnp.exp(zz - jnp.max(zz, axis=0, keepdims=True))
        y_soft = y / jnp.sum(y, axis=0, keepdims=True)
        mask_ref[...] = (y_soft > THRESHOLD).astype(jnp.int8)


@jax.jit
def kernel(target_emb, candidate_emb, in_proj_weight, in_proj_bias,
           out_proj_weight, out_proj_bias):
    t = target_emb[0]
    wq = in_proj_weight[:E]
    wk = in_proj_weight[E:2 * E]
    wv = in_proj_weight[2 * E:]
    wo = out_proj_weight
    bq = in_proj_bias[:E].reshape(1, E)
    bv = in_proj_bias[2 * E:].reshape(1, E)
    bo = out_proj_bias.reshape(1, E)

    u = jax.random.uniform(jax.random.key(42), (N, 1, S),
                           minval=1e-10, maxval=1.0)
    g = -jnp.log(-jnp.log(u))
    g_t = g[:, 0, :].T

    nb = S // SB
    out, mask = pl.pallas_call(
        _attn_kernel,
        grid=(nb,),
        in_specs=[
            pl.BlockSpec((N, E), lambda i: (0, 0)),
            pl.BlockSpec((E, E), lambda i: (0, 0)),
            pl.BlockSpec((1, E), lambda i: (0, 0)),
            pl.BlockSpec((E, E), lambda i: (0, 0)),
            pl.BlockSpec((E, E), lambda i: (0, 0)),
            pl.BlockSpec((1, E), lambda i: (0, 0)),
            pl.BlockSpec((E, E), lambda i: (0, 0)),
            pl.BlockSpec((1, E), lambda i: (0, 0)),
            pl.BlockSpec((S, N), lambda i: (0, 0)),
            pl.BlockSpec((SB, N, E), lambda i: (i, 0, 0)),
        ],
        out_specs=[
            pl.BlockSpec((N, E), lambda i: (0, 0)),
            pl.BlockSpec((S, N), lambda i: (0, 0)),
        ],
        out_shape=[
            jax.ShapeDtypeStruct((N, E), jnp.float32),
            jax.ShapeDtypeStruct((S, N), jnp.int8),
        ],
        scratch_shapes=[
            pltpu.VMEM((N, E), jnp.float32),
            pltpu.VMEM((1, N), jnp.float32),
            pltpu.VMEM((N, E), jnp.float32),
            pltpu.VMEM((S, N), jnp.float32),
        ],
    )(t, wq, bq, wk, wv, bv, wo, bo, g_t, candidate_emb)

    attn_output = out
    candidate_mask = mask.T.astype(jnp.bool_).reshape(N, 1, S)
    return attn_output, candidate_mask


# speedup vs baseline: 1.4616x; 1.0845x over previous
import math

import jax
import jax.numpy as jnp
from jax import lax
from jax.experimental import pallas as pl
from jax.experimental.pallas import tpu as pltpu

E = 128
N = 64
S = 8192
SB = 256
TAU = 1.0
THRESHOLD = 0.2

_HI = lax.Precision.HIGHEST


def _col_bcast(row, ones_row):
    return lax.dot_general(row, ones_row, (((0,), (0,)), ((), ())),
                           preferred_element_type=jnp.float32, precision=_HI)


def _attn_kernel(t_ref, wq_ref, bq_ref, wk_ref, wv_ref, bv_ref, wo_ref,
                 bo_ref, g_ref, c_ref, out_ref, mask_ref,
                 a_ref, racc_ref, l_ref):
    i = pl.program_id(0)
    nb = pl.num_programs(0)
    scale = 1.0 / math.sqrt(E)

    @pl.when(i == 0)
    def _init():
        q = lax.dot_general(t_ref[...], wq_ref[...], (((1,), (1,)), ((), ())),
                            preferred_element_type=jnp.float32,
                            precision=_HI) + bq_ref[...]
        a_ref[...] = lax.dot_general(q * scale, wk_ref[...],
                                     (((1,), (0,)), ((), ())),
                                     preferred_element_type=jnp.float32,
                                     precision=_HI)
        racc_ref[...] = jnp.zeros((N, N * E), jnp.float32)

    c = c_ref[...]
    logits = jnp.sum(c * a_ref[...][None, :, :], axis=-1)
    l_ref[pl.ds(i * SB, SB), :] = logits

    p = jnp.exp(logits)

    c3 = c.reshape(SB, N * E).astype(jnp.bfloat16)
    p16 = p.astype(jnp.bfloat16)
    racc_ref[...] += lax.dot_general(p16, c3, (((0,), (0,)), ((), ())),
                                     preferred_element_type=jnp.float32)

    @pl.when(i == nb - 1)
    def _finish():
        el = jnp.exp(l_ref[...])
        inv_d = 1.0 / jnp.sum(el, axis=0, keepdims=True)
        ones_row = jnp.ones((1, E), jnp.float32)
        r = racc_ref[...]
        cv_u = jnp.concatenate(
            [r[n:n + 1, n * E:(n + 1) * E] for n in range(N)], axis=0)
        cv = cv_u * _col_bcast(inv_d, ones_row)
        ctx = lax.dot_general(cv, wv_ref[...], (((1,), (1,)), ((), ())),
                              preferred_element_type=jnp.float32,
                              precision=_HI) + bv_ref[...]
        out_ref[...] = lax.dot_general(ctx, wo_ref[...],
                                       (((1,), (1,)), ((), ())),
                                       preferred_element_type=jnp.float32,
                                       precision=_HI) + bo_ref[...]
        attn = el * inv_d
        zz = (attn + g_ref[...]) / TAU
        y = jnp.exp(zz - jnp.max(zz, axis=0, keepdims=True))
        y_soft = y / jnp.sum(y, axis=0, keepdims=True)
        mask_ref[...] = (y_soft > THRESHOLD).astype(jnp.int8)


@jax.jit
def kernel(target_emb, candidate_emb, in_proj_weight, in_proj_bias,
           out_proj_weight, out_proj_bias):
    t = target_emb[0]
    wq = in_proj_weight[:E]
    wk = in_proj_weight[E:2 * E]
    wv = in_proj_weight[2 * E:]
    wo = out_proj_weight
    bq = in_proj_bias[:E].reshape(1, E)
    bv = in_proj_bias[2 * E:].reshape(1, E)
    bo = out_proj_bias.reshape(1, E)

    u = jax.random.uniform(jax.random.key(42), (N, 1, S),
                           minval=1e-10, maxval=1.0)
    g = -jnp.log(-jnp.log(u))
    g_t = g[:, 0, :].T

    nb = S // SB
    out, mask = pl.pallas_call(
        _attn_kernel,
        grid=(nb,),
        in_specs=[
            pl.BlockSpec((N, E), lambda i: (0, 0)),
            pl.BlockSpec((E, E), lambda i: (0, 0)),
            pl.BlockSpec((1, E), lambda i: (0, 0)),
            pl.BlockSpec((E, E), lambda i: (0, 0)),
            pl.BlockSpec((E, E), lambda i: (0, 0)),
            pl.BlockSpec((1, E), lambda i: (0, 0)),
            pl.BlockSpec((E, E), lambda i: (0, 0)),
            pl.BlockSpec((1, E), lambda i: (0, 0)),
            pl.BlockSpec((S, N), lambda i: (0, 0)),
            pl.BlockSpec((SB, N, E), lambda i: (i, 0, 0)),
        ],
        out_specs=[
            pl.BlockSpec((N, E), lambda i: (0, 0)),
            pl.BlockSpec((S, N), lambda i: (0, 0)),
        ],
        out_shape=[
            jax.ShapeDtypeStruct((N, E), jnp.float32),
            jax.ShapeDtypeStruct((S, N), jnp.int8),
        ],
        scratch_shapes=[
            pltpu.VMEM((N, E), jnp.float32),
            pltpu.VMEM((N, N * E), jnp.float32),
            pltpu.VMEM((S, N), jnp.float32),
        ],
    )(t, wq, bq, wk, wv, bv, wo, bo, g_t, candidate_emb)

    attn_output = out
    candidate_mask = mask.T.astype(jnp.bool_).reshape(N, 1, S)
    return attn_output, candidate_mask


# speedup vs baseline: 1.6807x; 1.1500x over previous
import math

import jax
import jax.numpy as jnp
from jax import lax
from jax.experimental import pallas as pl
from jax.experimental.pallas import tpu as pltpu

E = 128
N = 64
S = 8192
SB = 256
TAU = 1.0
THRESHOLD = 0.2

_HI = lax.Precision.HIGHEST


def _col_bcast(row, ones_row):
    return lax.dot_general(row, ones_row, (((0,), (0,)), ((), ())),
                           preferred_element_type=jnp.float32, precision=_HI)


def _attn_kernel(t_ref, wq_ref, bq_ref, wk_ref, wv_ref, bv_ref, wo_ref,
                 bo_ref, g_ref, c_ref, out_ref, mask_ref,
                 a_ref, racc_ref, l_ref):
    i = pl.program_id(0)
    nb = pl.num_programs(0)
    scale = 1.0 / math.sqrt(E)

    @pl.when(i == 0)
    def _init():
        q = lax.dot_general(t_ref[...], wq_ref[...], (((1,), (1,)), ((), ())),
                            preferred_element_type=jnp.float32,
                            precision=_HI) + bq_ref[...]
        a_ref[...] = lax.dot_general(q * scale, wk_ref[...],
                                     (((1,), (0,)), ((), ())),
                                     preferred_element_type=jnp.float32,
                                     precision=_HI)
        racc_ref[...] = jnp.zeros((E, N), jnp.float32)

    c = c_ref[...]
    logits = jnp.sum(c * a_ref[...][None, :, :], axis=-1)
    l_ref[pl.ds(i * SB, SB), :] = logits

    p = jnp.exp(logits)

    row = lax.broadcasted_iota(jnp.int32, (1, N, N), 1)
    col = lax.broadcasted_iota(jnp.int32, (1, N, N), 2)
    eye = (row == col).astype(jnp.bfloat16)
    p16 = p.astype(jnp.bfloat16)
    p2 = (eye * p16[:, None, :]).reshape(SB * N, N)
    c2 = c.reshape(SB * N, E).astype(jnp.bfloat16)
    racc_ref[...] += lax.dot_general(c2, p2, (((0,), (0,)), ((), ())),
                                     preferred_element_type=jnp.float32)

    @pl.when(i == nb - 1)
    def _finish():
        el = jnp.exp(l_ref[...])
        inv_d = 1.0 / jnp.sum(el, axis=0, keepdims=True)
        cv = jnp.transpose(racc_ref[...] * inv_d)
        ctx = lax.dot_general(cv, wv_ref[...], (((1,), (1,)), ((), ())),
                              preferred_element_type=jnp.float32,
                              precision=_HI) + bv_ref[...]
        out_ref[...] = lax.dot_general(ctx, wo_ref[...],
                                       (((1,), (1,)), ((), ())),
                                       preferred_element_type=jnp.float32,
                                       precision=_HI) + bo_ref[...]
        attn = el * inv_d
        zz = (attn + g_ref[...]) / TAU
        y = jnp.exp(zz - jnp.max(zz, axis=0, keepdims=True))
        y_soft = y / jnp.sum(y, axis=0, keepdims=True)
        mask_ref[...] = (y_soft > THRESHOLD).astype(jnp.int8)


@jax.jit
def kernel(target_emb, candidate_emb, in_proj_weight, in_proj_bias,
           out_proj_weight, out_proj_bias):
    t = target_emb[0]
    wq = in_proj_weight[:E]
    wk = in_proj_weight[E:2 * E]
    wv = in_proj_weight[2 * E:]
    wo = out_proj_weight
    bq = in_proj_bias[:E].reshape(1, E)
    bv = in_proj_bias[2 * E:].reshape(1, E)
    bo = out_proj_bias.reshape(1, E)

    u = jax.random.uniform(jax.random.key(42), (N, 1, S),
                           minval=1e-10, maxval=1.0)
    g = -jnp.log(-jnp.log(u))
    g_t = g[:, 0, :].T

    nb = S // SB
    out, mask = pl.pallas_call(
        _attn_kernel,
        grid=(nb,),
        in_specs=[
            pl.BlockSpec((N, E), lambda i: (0, 0)),
            pl.BlockSpec((E, E), lambda i: (0, 0)),
            pl.BlockSpec((1, E), lambda i: (0, 0)),
            pl.BlockSpec((E, E), lambda i: (0, 0)),
            pl.BlockSpec((E, E), lambda i: (0, 0)),
            pl.BlockSpec((1, E), lambda i: (0, 0)),
            pl.BlockSpec((E, E), lambda i: (0, 0)),
            pl.BlockSpec((1, E), lambda i: (0, 0)),
            pl.BlockSpec((S, N), lambda i: (0, 0)),
            pl.BlockSpec((SB, N, E), lambda i: (i, 0, 0)),
        ],
        out_specs=[
            pl.BlockSpec((N, E), lambda i: (0, 0)),
            pl.BlockSpec((S, N), lambda i: (0, 0)),
        ],
        out_shape=[
            jax.ShapeDtypeStruct((N, E), jnp.float32),
            jax.ShapeDtypeStruct((S, N), jnp.int8),
        ],
        scratch_shapes=[
            pltpu.VMEM((N, E), jnp.float32),
            pltpu.VMEM((E, N), jnp.float32),
            pltpu.VMEM((S, N), jnp.float32),
        ],
    )(t, wq, bq, wk, wv, bv, wo, bo, g_t, candidate_emb)

    attn_output = out
    candidate_mask = mask.T.astype(jnp.bool_).reshape(N, 1, S)
    return attn_output, candidate_mask


# speedup vs baseline: 1.7108x; 1.0179x over previous
import math

import jax
import jax.numpy as jnp
from jax import lax
from jax.experimental import pallas as pl
from jax.experimental.pallas import tpu as pltpu

E = 128
N = 64
S = 8192
SB = 512
TAU = 1.0
THRESHOLD = 0.2

_HI = lax.Precision.HIGHEST


def _col_bcast(row, ones_row):
    return lax.dot_general(row, ones_row, (((0,), (0,)), ((), ())),
                           preferred_element_type=jnp.float32, precision=_HI)


def _attn_kernel(t_ref, wq_ref, bq_ref, wk_ref, wv_ref, bv_ref, wo_ref,
                 bo_ref, g_ref, c_ref, out_ref, mask_ref,
                 a_ref, racc_ref, l_ref):
    i = pl.program_id(0)
    nb = pl.num_programs(0)
    scale = 1.0 / math.sqrt(E)

    @pl.when(i == 0)
    def _init():
        q = lax.dot_general(t_ref[...], wq_ref[...], (((1,), (1,)), ((), ())),
                            preferred_element_type=jnp.float32,
                            precision=_HI) + bq_ref[...]
        a_ref[...] = lax.dot_general(q * scale, wk_ref[...],
                                     (((1,), (0,)), ((), ())),
                                     preferred_element_type=jnp.float32,
                                     precision=_HI)
        racc_ref[...] = jnp.zeros((E, N), jnp.float32)

    c = c_ref[...]
    logits = jnp.sum(c * a_ref[...][None, :, :], axis=-1)
    l_ref[pl.ds(i * SB, SB), :] = logits

    p = jnp.exp(logits)

    row = lax.broadcasted_iota(jnp.int32, (1, N, N), 1)
    col = lax.broadcasted_iota(jnp.int32, (1, N, N), 2)
    eye = (row == col).astype(jnp.bfloat16)
    p16 = p.astype(jnp.bfloat16)
    p2 = (eye * p16[:, None, :]).reshape(SB * N, N)
    c2 = c.reshape(SB * N, E).astype(jnp.bfloat16)
    racc_ref[...] += lax.dot_general(c2, p2, (((0,), (0,)), ((), ())),
                                     preferred_element_type=jnp.float32)

    @pl.when(i == nb - 1)
    def _finish():
        el = jnp.exp(l_ref[...])
        inv_d = 1.0 / jnp.sum(el, axis=0, keepdims=True)
        cv = jnp.transpose(racc_ref[...] * inv_d)
        ctx = lax.dot_general(cv, wv_ref[...], (((1,), (1,)), ((), ())),
                              preferred_element_type=jnp.float32,
                              precision=_HI) + bv_ref[...]
        out_ref[...] = lax.dot_general(ctx, wo_ref[...],
                                       (((1,), (1,)), ((), ())),
                                       preferred_element_type=jnp.float32,
                                       precision=_HI) + bo_ref[...]
        attn = el * inv_d
        zz = (attn + g_ref[...]) / TAU
        y = jnp.exp(zz - jnp.max(zz, axis=0, keepdims=True))
        y_soft = y / jnp.sum(y, axis=0, keepdims=True)
        mask_ref[...] = (y_soft > THRESHOLD).astype(jnp.int8)


@jax.jit
def kernel(target_emb, candidate_emb, in_proj_weight, in_proj_bias,
           out_proj_weight, out_proj_bias):
    t = target_emb[0]
    wq = in_proj_weight[:E]
    wk = in_proj_weight[E:2 * E]
    wv = in_proj_weight[2 * E:]
    wo = out_proj_weight
    bq = in_proj_bias[:E].reshape(1, E)
    bv = in_proj_bias[2 * E:].reshape(1, E)
    bo = out_proj_bias.reshape(1, E)

    u = jax.random.uniform(jax.random.key(42), (N, 1, S),
                           minval=1e-10, maxval=1.0)
    g = -jnp.log(-jnp.log(u))
    g_t = g[:, 0, :].T

    nb = S // SB
    out, mask = pl.pallas_call(
        _attn_kernel,
        grid=(nb,),
        in_specs=[
            pl.BlockSpec((N, E), lambda i: (0, 0)),
            pl.BlockSpec((E, E), lambda i: (0, 0)),
            pl.BlockSpec((1, E), lambda i: (0, 0)),
            pl.BlockSpec((E, E), lambda i: (0, 0)),
            pl.BlockSpec((E, E), lambda i: (0, 0)),
            pl.BlockSpec((1, E), lambda i: (0, 0)),
            pl.BlockSpec((E, E), lambda i: (0, 0)),
            pl.BlockSpec((1, E), lambda i: (0, 0)),
            pl.BlockSpec((S, N), lambda i: (0, 0)),
            pl.BlockSpec((SB, N, E), lambda i: (i, 0, 0)),
        ],
        out_specs=[
            pl.BlockSpec((N, E), lambda i: (0, 0)),
            pl.BlockSpec((S, N), lambda i: (0, 0)),
        ],
        out_shape=[
            jax.ShapeDtypeStruct((N, E), jnp.float32),
            jax.ShapeDtypeStruct((S, N), jnp.int8),
        ],
        scratch_shapes=[
            pltpu.VMEM((N, E), jnp.float32),
            pltpu.VMEM((E, N), jnp.float32),
            pltpu.VMEM((S, N), jnp.float32),
        ],
    )(t, wq, bq, wk, wv, bv, wo, bo, g_t, candidate_emb)

    attn_output = out
    candidate_mask = mask.T.astype(jnp.bool_).reshape(N, 1, S)
    return attn_output, candidate_mask
